# baseline (speedup 1.0000x reference)
import functools

import jax
import jax.numpy as jnp
from jax import lax
from jax.experimental import pallas as pl
from jax.experimental.pallas import tpu as pltpu
from jax.experimental.pallas import tpu_sc as plsc

_BATCH = 16384
_D = 32
_LANES = 16
_CHUNK = 128

_LOG1P_COEFFS = (
    8.184459776572339e-10,
    0.9999997947731366,
    -0.499991422832033,
    0.3331919078312559,
    -0.24878424198822732,
    0.19375165838711964,
    -0.14586473667566108,
    0.09548566247208555,
    -0.04766548569248698,
    0.015341897539328783,
    -0.002317854668945074,
)


def _make_sc_loss():
    mesh = plsc.VectorSubcoreMesh(core_axis_name="c", subcore_axis_name="s")
    nc = mesh.num_cores
    nw = nc * mesh.num_subcores
    bpw = _BATCH // nw
    nch = bpw // _CHUNK
    gpc = _CHUNK // _LANES

    @functools.partial(
        pl.kernel,
        out_type=jax.ShapeDtypeStruct((nc, _LANES), jnp.float32),
        mesh=mesh,
        scratch_types=[
            pltpu.VMEM((nch, _CHUNK), jnp.int32),
            pltpu.VMEM((nch, _CHUNK), jnp.int32),
            pltpu.VMEM((bpw, _D), jnp.float32),
            pltpu.VMEM((bpw, _D), jnp.float32),
            pltpu.VMEM((3 * _LANES,), jnp.float32),
            pltpu.VMEM((bpw,), jnp.float32),
            pltpu.VMEM((_LANES,), jnp.float32),
            pltpu.VMEM((_LANES, _LANES), jnp.float32),
            pltpu.VMEM_SHARED((_LANES, _LANES), jnp.float32),
            pltpu.SemaphoreType.DMA,
        ],
        compiler_params=pltpu.CompilerParams(
            needs_layout_passes=False, use_tc_tiling_on_sc=False),
    )
    def sc_loss(user_hbm, item_hbm, lab_hbm, ut_hbm, it_hbm, wb_hbm, out_hbm,
                idx_u, idx_v, u_rows, v_rows, wb_v, lab_v, part_v, allp_v,
                shared, sem):
        cid = lax.axis_index("c")
        sid = lax.axis_index("s")
        wid = sid * nc + cid

        pltpu.sync_copy(user_hbm.at[pl.ds(wid * nch, nch)], idx_u)
        pltpu.sync_copy(item_hbm.at[pl.ds(wid * nch, nch)], idx_v)
        pltpu.sync_copy(wb_hbm, wb_v)
        pltpu.sync_copy(lab_hbm.at[pl.ds(wid * bpw, bpw)], lab_v)

        copies = []
        for j in range(nch):
            dst = pl.ds(j * _CHUNK, _CHUNK)
            copies.append(
                (pltpu.async_copy(ut_hbm.at[idx_u.at[j]], u_rows.at[dst], sem),
                 pltpu.async_copy(it_hbm.at[idx_v.at[j]], v_rows.at[dst], sem)))

        w_lo = wb_v[pl.ds(0, _LANES)]
        w_hi = wb_v[pl.ds(_LANES, _LANES)]
        b_s = wb_v[pl.ds(2 * _LANES, _LANES)][0]
        wds = ([w_lo[d] for d in range(_LANES)]
               + [w_hi[d] for d in range(_LANES)])
        lane = lax.iota(jnp.int32, _LANES)

        def group_body(g, bce):
            rows = g * _LANES + lane
            acc = jnp.zeros((_LANES,), jnp.float32)
            for d in range(_D):
                cols = jnp.full((_LANES,), d, jnp.int32)
                uu = plsc.load_gather(u_rows, [rows, cols])
                vv = plsc.load_gather(v_rows, [rows, cols])
                acc = acc + (uu * vv) * wds[d]
            x = acc + b_s
            t = lab_v[pl.ds(g * _LANES, _LANES)]
            e = jnp.exp(-jnp.abs(x))
            sp = jnp.full((_LANES,), _LOG1P_COEFFS[-1], jnp.float32)
            for c in reversed(_LOG1P_COEFFS[:-1]):
                sp = sp * e + c
            z = jnp.maximum(x, 0.0) - x * t + sp
            return bce + z

        bce = jnp.zeros((_LANES,), jnp.float32)
        for j in range(nch):
            cu, cv = copies[j]
            cu.wait()
            cv.wait()
            bce = lax.fori_loop(j * gpc, (j + 1) * gpc, group_body, bce)

        part_v[...] = bce * (1.0 / _BATCH)
        pltpu.sync_copy(part_v, shared.at[sid])
        plsc.subcore_barrier()

        @pl.when(sid == 0)
        def _():
            pltpu.sync_copy(shared, allp_v)
            total = allp_v[0]
            for r in range(1, _LANES):
                total = total + allp_v[r]
            part_v[...] = total
            pltpu.sync_copy(part_v, out_hbm.at[cid])

    return sc_loss


_sc_loss = _make_sc_loss()


def kernel(user, item, label, user_table, item_table, W, b):
    wb = jnp.concatenate(
        [W.reshape(-1), b.reshape(-1),
         jnp.zeros((_LANES - 1,), jnp.float32)])
    partials = _sc_loss(user.reshape(-1, _CHUNK), item.reshape(-1, _CHUNK),
                        label, user_table, item_table, wb)
    return jnp.sum(partials)
